# baseline (speedup 1.0000x reference)
import functools

import jax
import jax.numpy as jnp
from jax import lax
from jax.experimental import pallas as pl
from jax.experimental.pallas import tpu as pltpu
from jax.experimental.pallas import tpu_sc as plsc

B = 4096
DIM = 64
TOK = 8
NC, NS, L = 2, 16, 16
NW = NC * NS
TW = B // NW
C = 32
MARGIN = 1.0


def _rsqrt(x):
    i = plsc.bitcast(x, jnp.int32)
    y = plsc.bitcast(jnp.int32(0x5F3759DF) - (i >> 1), jnp.float32)
    for _ in range(3):
        y = y * (1.5 - 0.5 * x * y * y)
    return y


def _sc_body(hs, rs, ts, ent, rel, wemb, etok, rtok,
             loss_o, pd_o, nd_o,
             ih, ir, it, eh_v, et_v, rr_v, th_v, tt_v, tr_v,
             wih, wir, wit, wh_v, wr_v, wt_v, dv, lv,
             sem_e, sem_t, sem_w):
    wid = lax.axis_index("s") * NC + lax.axis_index("c")
    iota = lax.broadcasted_iota(jnp.int32, (L,), 0)
    zf = jnp.zeros((L,), jnp.float32)

    for set_id in range(2):
        base = pl.multiple_of(set_id * B + wid * TW, 8)
        dst = pl.ds(set_id * TW, TW)
        pltpu.sync_copy(hs.at[pl.ds(base, TW)], ih.at[dst])
        pltpu.sync_copy(rs.at[pl.ds(base, TW)], ir.at[dst])
        pltpu.sync_copy(ts.at[pl.ds(base, TW)], it.at[dst])

    def one_pass(sp, _):
        off = pl.multiple_of(sp * C, 8)
        sl = pl.ds(off, C)
        ce1 = pltpu.async_copy(ent.at[ih.at[sl]], eh_v, sem_e)
        ce2 = pltpu.async_copy(ent.at[it.at[sl]], et_v, sem_e)
        ce3 = pltpu.async_copy(rel.at[ir.at[sl]], rr_v, sem_e)
        ct1 = pltpu.async_copy(etok.at[ih.at[sl]], th_v, sem_t)
        ct2 = pltpu.async_copy(etok.at[it.at[sl]], tt_v, sem_t)
        ct3 = pltpu.async_copy(rtok.at[ir.at[sl]], tr_v, sem_t)
        ct1.wait()
        ct2.wait()
        ct3.wait()

        word_copies = []
        for tokv, wi, wv in ((th_v, wih, wh_v), (tr_v, wir, wr_v),
                             (tt_v, wit, wt_v)):
            for k in range(C * TOK // L):
                p16 = k * L + iota
                tvec = plsc.load_gather(tokv, [p16 // TOK, p16 % TOK])
                wi[k // 8, pl.ds((k % 8) * L, L)] = tvec
            for j in range(2):
                word_copies.append(pltpu.async_copy(
                    wemb.at[wi.at[j]], wv.at[pl.ds(j * 128, 128)], sem_w))
        ce1.wait()
        ce2.wait()
        ce3.wait()
        for cw in word_copies:
            cw.wait()

        def one_group(g, _):
            rows = g * L + iota

            def p1(d, carry):
                sh, st = carry
                cd = lax.broadcast(d, (L,))
                a = plsc.load_gather(eh_v, [rows, cd])
                b = plsc.load_gather(et_v, [rows, cd])
                return sh + a * a, st + b * b

            sh, st = lax.fori_loop(0, DIM, p1, (zf, zf))
            inv_h = _rsqrt(sh)
            inv_t = _rsqrt(st)

            def p2(d, acc):
                cd = lax.broadcast(d, (L,))
                eh = plsc.load_gather(eh_v, [rows, cd])
                et = plsc.load_gather(et_v, [rows, cd])
                rr = plsc.load_gather(rr_v, [rows, cd])
                wh = zf
                wr = zf
                wt = zf
                for j in range(TOK):
                    wrow = rows * TOK + j
                    wh = wh + plsc.load_gather(wh_v, [wrow, cd])
                    wr = wr + plsc.load_gather(wr_v, [wrow, cd])
                    wt = wt + plsc.load_gather(wt_v, [wrow, cd])
                v = (eh * inv_h * (wh * 0.125) + rr * (wr * 0.125)
                     - et * inv_t * (wt * 0.125))
                return acc + jnp.abs(v)

            dist = lax.fori_loop(0, DIM, p2, zf)
            dv[pl.ds(off + g * L, L)] = dist
            return 0

        lax.fori_loop(0, C // L, one_group, 0)
        return 0

    lax.fori_loop(0, 2 * TW // C, one_pass, 0)

    for k in range(TW // L):
        s = pl.ds(k * L, L)
        lv[s] = jnp.maximum(dv[s] - dv[pl.ds(TW + k * L, L)] + MARGIN, 0.0)

    obase = pl.multiple_of(wid * TW, 8)
    pltpu.sync_copy(lv, loss_o.at[pl.ds(obase, TW)])
    pltpu.sync_copy(dv.at[pl.ds(0, TW)], pd_o.at[pl.ds(obase, TW)])
    pltpu.sync_copy(dv.at[pl.ds(TW, TW)], nd_o.at[pl.ds(obase, TW)])


@jax.jit
def _sc_call(hs, rs, ts, ent, rel, wemb, etok, rtok):
    mesh = plsc.VectorSubcoreMesh(core_axis_name="c", subcore_axis_name="s")
    f32 = jnp.float32
    run = functools.partial(
        pl.kernel,
        out_type=[jax.ShapeDtypeStruct((B,), f32)] * 3,
        mesh=mesh,
        compiler_params=pltpu.CompilerParams(
            use_tc_tiling_on_sc=False, needs_layout_passes=False),
        scratch_types=[
            pltpu.VMEM((2 * TW,), jnp.int32),
            pltpu.VMEM((2 * TW,), jnp.int32),
            pltpu.VMEM((2 * TW,), jnp.int32),
            pltpu.VMEM((C, DIM), f32),
            pltpu.VMEM((C, DIM), f32),
            pltpu.VMEM((C, DIM), f32),
            pltpu.VMEM((C, TOK), jnp.int32),
            pltpu.VMEM((C, TOK), jnp.int32),
            pltpu.VMEM((C, TOK), jnp.int32),
            pltpu.VMEM((2, 128), jnp.int32),
            pltpu.VMEM((2, 128), jnp.int32),
            pltpu.VMEM((2, 128), jnp.int32),
            pltpu.VMEM((C * TOK, DIM), f32),
            pltpu.VMEM((C * TOK, DIM), f32),
            pltpu.VMEM((C * TOK, DIM), f32),
            pltpu.VMEM((2 * TW,), f32),
            pltpu.VMEM((TW,), f32),
            pltpu.SemaphoreType.DMA,
            pltpu.SemaphoreType.DMA,
            pltpu.SemaphoreType.DMA,
        ],
    )(_sc_body)
    return run(hs, rs, ts, ent, rel, wemb, etok, rtok)


def kernel(positive_triplets, negative_triplets, entities_emb, relations_emb,
           word_emb, entity_token_ids, relation_token_ids):
    hs = jnp.concatenate([positive_triplets[:, 0], negative_triplets[:, 0]])
    rs = jnp.concatenate([positive_triplets[:, 1], negative_triplets[:, 1]])
    ts = jnp.concatenate([positive_triplets[:, 2], negative_triplets[:, 2]])
    loss, pd, nd = _sc_call(hs, rs, ts, entities_emb, relations_emb, word_emb,
                            entity_token_ids, relation_token_ids)
    return (loss, pd, nd)


# speedup vs baseline: 2.2413x; 2.2413x over previous
import functools

import jax
import jax.numpy as jnp
from jax import lax
from jax.experimental import pallas as pl
from jax.experimental.pallas import tpu as pltpu
from jax.experimental.pallas import tpu_sc as plsc

B = 4096
DIM = 64
TOK = 8
NC, NS, L = 2, 16, 16
NW = NC * NS
TW = B // NW
G = TW // L
MARGIN = 1.0


def _rsqrt(x):
    i = plsc.bitcast(x, jnp.int32)
    y = plsc.bitcast(jnp.int32(0x5F3759DF) - (i >> 1), jnp.float32)
    for _ in range(3):
        y = y * (1.5 - 0.5 * x * y * y)
    return y


def _sc_body(eh_f, et_f, rr_f, th_f, tt_f, tr_f, wemb2,
             loss_o, pd_o, nd_o,
             ehb, etb, rrb, thb, ttb, trb,
             wring, widx, parb, tm, dv, lv,
             sem_w):
    wid = lax.axis_index("s") * NC + lax.axis_index("c")
    iota = lax.broadcasted_iota(jnp.int32, (L,), 0)
    zf = jnp.zeros((L,), jnp.float32)
    tok_bufs = (thb, trb, ttb)

    for set_id in range(2):
        tbase = set_id * B + wid * TW
        pltpu.sync_copy(eh_f.at[pl.ds(pl.multiple_of(tbase * DIM, 8), TW * DIM)], ehb)
        pltpu.sync_copy(et_f.at[pl.ds(pl.multiple_of(tbase * DIM, 8), TW * DIM)], etb)
        pltpu.sync_copy(rr_f.at[pl.ds(pl.multiple_of(tbase * DIM, 8), TW * DIM)], rrb)
        pltpu.sync_copy(th_f.at[pl.ds(pl.multiple_of(tbase * TOK, 8), TW * TOK)], thb)
        pltpu.sync_copy(tt_f.at[pl.ds(pl.multiple_of(tbase * TOK, 8), TW * TOK)], ttb)
        pltpu.sync_copy(tr_f.at[pl.ds(pl.multiple_of(tbase * TOK, 8), TW * TOK)], trb)

        def one_group(g, _):
            for t, tkb in enumerate(tok_bufs):
                for j in range(TOK):
                    tv = plsc.load_gather(tkb, [(g * L + iota) * TOK + j])
                    widx[pl.ds((t * TOK + j) * L, L)] = tv >> 1
                    parb[pl.ds((t * TOK + j) * L, L)] = (tv & 1) * DIM
            copies = [
                pltpu.async_copy(wemb2.at[widx.at[pl.ds(t * 128, 128)]],
                                 wring.at[pl.ds(t * 128, 128)], sem_w)
                for t in range(3)
            ]
            for cp in copies:
                cp.wait()

            def tm_zero(i, _):
                tm[pl.ds(i * L, L)] = zf
                return 0

            lax.fori_loop(0, 3 * DIM, tm_zero, 0)

            def tm_acc(k, _):
                t = k // TOK
                parv = parb[pl.ds(k * L, L)]
                rowv = k * L + iota
                for d in range(DIM):
                    x = plsc.load_gather(wring, [rowv, parv + d])
                    plsc.addupdate(tm.at[pl.ds((t * DIM + d) * L, L)], x)
                return 0

            lax.fori_loop(0, 3 * TOK, tm_acc, 0)

            rows0 = (g * L + iota) * DIM

            def p1(d, carry):
                sh, st = carry
                a = plsc.load_gather(ehb, [rows0 + d])
                b = plsc.load_gather(etb, [rows0 + d])
                return sh + a * a, st + b * b

            sh, st = lax.fori_loop(0, DIM, p1, (zf, zf))
            inv_h = _rsqrt(sh) * 0.125
            inv_t = _rsqrt(st) * 0.125

            def p2(d, acc):
                eh = plsc.load_gather(ehb, [rows0 + d])
                et = plsc.load_gather(etb, [rows0 + d])
                rr = plsc.load_gather(rrb, [rows0 + d])
                th = tm[pl.ds((0 * DIM + d) * L, L)]
                tr = tm[pl.ds((1 * DIM + d) * L, L)]
                tt = tm[pl.ds((2 * DIM + d) * L, L)]
                v = eh * inv_h * th + rr * (tr * 0.125) - et * inv_t * tt
                return acc + jnp.abs(v)

            dist = lax.fori_loop(0, DIM, p2, zf)
            dv[pl.ds(set_id * TW + g * L, L)] = dist
            return 0

        lax.fori_loop(0, G, one_group, 0)

    for k in range(TW // L):
        s = pl.ds(k * L, L)
        lv[s] = jnp.maximum(dv[s] - dv[pl.ds(TW + k * L, L)] + MARGIN, 0.0)

    obase = pl.multiple_of(wid * TW, 8)
    pltpu.sync_copy(lv, loss_o.at[pl.ds(obase, TW)])
    pltpu.sync_copy(dv.at[pl.ds(0, TW)], pd_o.at[pl.ds(obase, TW)])
    pltpu.sync_copy(dv.at[pl.ds(TW, TW)], nd_o.at[pl.ds(obase, TW)])


@jax.jit
def _sc_call(eh_f, et_f, rr_f, th_f, tt_f, tr_f, wemb2):
    mesh = plsc.VectorSubcoreMesh(core_axis_name="c", subcore_axis_name="s")
    f32 = jnp.float32
    i32 = jnp.int32
    run = functools.partial(
        pl.kernel,
        out_type=[jax.ShapeDtypeStruct((B,), f32)] * 3,
        mesh=mesh,
        compiler_params=pltpu.CompilerParams(
            use_tc_tiling_on_sc=True, needs_layout_passes=False),
        scratch_types=[
            pltpu.VMEM((TW * DIM,), f32),
            pltpu.VMEM((TW * DIM,), f32),
            pltpu.VMEM((TW * DIM,), f32),
            pltpu.VMEM((TW * TOK,), i32),
            pltpu.VMEM((TW * TOK,), i32),
            pltpu.VMEM((TW * TOK,), i32),
            pltpu.VMEM((3 * TOK * L, 128), f32),
            pltpu.VMEM((3 * TOK * L,), i32),
            pltpu.VMEM((3 * TOK * L,), i32),
            pltpu.VMEM((3 * DIM * L,), f32),
            pltpu.VMEM((2 * TW,), f32),
            pltpu.VMEM((TW,), f32),
            pltpu.SemaphoreType.DMA,
        ],
    )(_sc_body)
    return run(eh_f, et_f, rr_f, th_f, tt_f, tr_f, wemb2)


def kernel(positive_triplets, negative_triplets, entities_emb, relations_emb,
           word_emb, entity_token_ids, relation_token_ids):
    hs = jnp.concatenate([positive_triplets[:, 0], negative_triplets[:, 0]])
    rs = jnp.concatenate([positive_triplets[:, 1], negative_triplets[:, 1]])
    ts = jnp.concatenate([positive_triplets[:, 2], negative_triplets[:, 2]])
    eh_f = entities_emb[hs].reshape(-1)
    et_f = entities_emb[ts].reshape(-1)
    rr_f = relations_emb[rs].reshape(-1)
    th_f = entity_token_ids[hs].reshape(-1)
    tt_f = entity_token_ids[ts].reshape(-1)
    tr_f = relation_token_ids[rs].reshape(-1)
    wemb2 = word_emb.reshape(16000, 128)
    loss, pd, nd = _sc_call(eh_f, et_f, rr_f, th_f, tt_f, tr_f, wemb2)
    return (loss, pd, nd)
